# Initial kernel scaffold; baseline (speedup 1.0000x reference)
import functools

import jax
import jax.numpy as jnp
from jax import lax
from jax.experimental import pallas as pl
from jax.experimental.pallas import tpu as pltpu
from jax.experimental.pallas import tpu_sc as plsc

BATCH = 16384
FIELDS = 26
N = BATCH * FIELDS
EMB = 64
NC = 2
NS = 16
NW = NC * NS
TPW = N // NW
BLK = 128
NB = TPW // BLK
L = 16


def _pt_matmul(lhs, rhs):
    def body(l_ref, r_ref, o_ref):
        o_ref[...] = jnp.dot(l_ref[...], r_ref[...],
                             preferred_element_type=jnp.float32)
    return pl.pallas_call(
        body,
        out_shape=jax.ShapeDtypeStruct((400, 3200), jnp.float32),
    )(lhs, rhs)


def _splat(v):
    return jnp.full((L,), v, dtype=jnp.int32)


_mesh = plsc.VectorSubcoreMesh(core_axis_name="c", subcore_axis_name="s")


@functools.partial(
    pl.kernel,
    mesh=_mesh,
    out_type=jax.ShapeDtypeStruct((N, EMB), jnp.float32),
    scratch_types=[
        pltpu.VMEM((TPW,), jnp.int32),
        pltpu.VMEM((100, 32), jnp.float32),
        pltpu.VMEM((2, 4, BLK), jnp.int32),
        pltpu.VMEM((2, BLK), jnp.int32),
        pltpu.VMEM((2, 4, BLK, 32), jnp.float32),
        pltpu.VMEM((2, BLK, EMB), jnp.float32),
        pltpu.SemaphoreType.DMA,
        pltpu.SemaphoreType.DMA,
        pltpu.SemaphoreType.DMA,
        pltpu.SemaphoreType.DMA,
    ],
)
def _sc_lookup(idx_hbm, pt_hbm, ct_hbm, out_hbm,
               idx_v, ct_v, gidx_v, i3_v, rows_v, out_v,
               gsem0, gsem1, osem0, osem1):
    gsems = (gsem0, gsem1)
    osems = (osem0, osem1)
    wid = lax.axis_index("s") * NC + lax.axis_index("c")
    base = wid * TPW

    pltpu.sync_copy(idx_hbm.at[pl.ds(base, TPW)], idx_v)
    pltpu.sync_copy(ct_hbm, ct_v)

    def make_idx(bb, slot):
        for v in range(BLK // L):
            iv = idx_v[pl.ds(bb * BLK + v * L, L)]
            f = iv.astype(jnp.float32) + 0.5
            i12 = (f * 0.01).astype(jnp.int32)
            i3 = iv - i12 * 100
            for j1 in range(4):
                gidx_v[slot, j1, pl.ds(v * L, L)] = i12 + j1 * 10000
            i3_v[slot, pl.ds(v * L, L)] = i3

    def fire(slot):
        for j1 in range(4):
            pltpu.async_copy(pt_hbm.at[gidx_v.at[slot, j1]],
                             rows_v.at[slot, j1], gsems[slot])

    def drain(slot):
        for j1 in range(4):
            pltpu.make_async_copy(pt_hbm.at[gidx_v.at[slot, j1]],
                                  rows_v.at[slot, j1], gsems[slot]).wait()

    def owait(slot):
        pltpu.make_async_copy(out_v.at[slot],
                              out_hbm.at[pl.ds(base, BLK)],
                              osems[slot]).wait()

    def compute(slot):
        def gbody(g, carry):
            tvec = lax.iota(jnp.int32, L) + g * L
            i3g = i3_v[slot, pl.ds(g * L, L)]
            cv = [plsc.load_gather(ct_v, [i3g, _splat(c)]) for c in range(32)]
            for jj in range(16):
                j1, j2 = jj // 4, jj % 4
                pv = [plsc.load_gather(rows_v.at[slot, j1],
                                       [tvec, _splat(j2 * 8 + r2)])
                      for r2 in range(8)]
                for j3 in range(4):
                    acc = pv[0] * cv[j3]
                    for r2 in range(1, 8):
                        acc = acc + pv[r2] * cv[r2 * 4 + j3]
                    plsc.store_scatter(out_v.at[slot],
                                       [tvec, _splat(jj * 4 + j3)], acc)
            return carry
        lax.fori_loop(0, BLK // L, gbody, 0)

    make_idx(0, 0)
    fire(0)

    def outer(i, carry):
        for par in range(2):
            bb = 2 * i + par

            @pl.when(bb + 1 < NB)
            def _prefetch():
                make_idx(bb + 1, 1 - par)
                fire(1 - par)

            drain(par)

            @pl.when(bb >= 2)
            def _wait_out():
                owait(par)

            compute(par)
            pltpu.async_copy(out_v.at[par],
                             out_hbm.at[pl.ds(base + bb * BLK, BLK)],
                             osems[par])
        return carry

    lax.fori_loop(0, NB // 2, outer, 0)
    owait(0)
    owait(1)


def kernel(indices, core0, core1, core2):
    lhs = jnp.transpose(core0[0], (1, 0, 2)).reshape(400, 8)
    rhs = core1.reshape(8, 3200)
    pt = _pt_matmul(lhs, rhs).reshape(40000, 32)
    ct = jnp.transpose(core2[:, :, :, 0], (1, 0, 2)).reshape(100, 32)
    idx = indices.reshape(-1)
    out = _sc_lookup(idx, pt, ct)
    return out.reshape(BATCH, FIELDS, EMB)


# speedup vs baseline: 9.6085x; 9.6085x over previous
import functools

import jax
import jax.numpy as jnp
from jax import lax
from jax.experimental import pallas as pl
from jax.experimental.pallas import tpu as pltpu
from jax.experimental.pallas import tpu_sc as plsc

BATCH = 16384
FIELDS = 26
N = BATCH * FIELDS
EMB = 64
NC = 2
NS = 16
NW = NC * NS
TPW = N // NW
BLK = 128
NB = TPW // BLK
L = 16


def _pt_matmul(lhs, rhs):
    def body(l_ref, r_ref, o_ref):
        o_ref[...] = jnp.dot(l_ref[...], r_ref[...],
                             preferred_element_type=jnp.float32)
    return pl.pallas_call(
        body,
        out_shape=jax.ShapeDtypeStruct((100, 12800), jnp.float32),
    )(lhs, rhs)


def _splat(v):
    return jnp.full((L,), v, dtype=jnp.int32)


_mesh = plsc.VectorSubcoreMesh(core_axis_name="c", subcore_axis_name="s")


@functools.partial(
    pl.kernel,
    mesh=_mesh,
    compiler_params=pltpu.CompilerParams(needs_layout_passes=False),
    out_type=jax.ShapeDtypeStruct((N * EMB,), jnp.float32),
    scratch_types=[
        pltpu.VMEM((TPW,), jnp.int32),
        pltpu.VMEM((3200,), jnp.float32),
        pltpu.VMEM((BLK,), jnp.int32),
        pltpu.VMEM((BLK,), jnp.int32),
        pltpu.VMEM((BLK,), jnp.int32),
        pltpu.VMEM((BLK,), jnp.int32),
        pltpu.VMEM((BLK, 128), jnp.float32),
        pltpu.VMEM((BLK, 128), jnp.float32),
        pltpu.VMEM((BLK * EMB,), jnp.float32),
        pltpu.VMEM((BLK * EMB,), jnp.float32),
        pltpu.SemaphoreType.DMA,
        pltpu.SemaphoreType.DMA,
        pltpu.SemaphoreType.DMA,
        pltpu.SemaphoreType.DMA,
    ],
)
def _sc_lookup(idx_hbm, pt_hbm, ct_hbm, out_hbm,
               idx_v, ct_v, gidx0, gidx1, i30, i31, rows0, rows1,
               out0, out1, gsem0, gsem1, osem0, osem1):
    gidxs = (gidx0, gidx1)
    i3s = (i30, i31)
    rows = (rows0, rows1)
    outs = (out0, out1)
    gsems = (gsem0, gsem1)
    osems = (osem0, osem1)
    wid = lax.axis_index("s") * NC + lax.axis_index("c")
    base = wid * TPW

    pltpu.sync_copy(idx_hbm.at[pl.ds(base, TPW)], idx_v)
    pltpu.sync_copy(ct_hbm, ct_v)

    def make_idx(bb, slot):
        for v in range(BLK // L):
            iv = idx_v[pl.ds(bb * BLK + v * L, L)]
            f = iv.astype(jnp.float32) + 0.5
            i12 = (f * 0.01).astype(jnp.int32)
            i3 = iv - i12 * 100
            gidxs[slot][pl.ds(v * L, L)] = i12
            i3s[slot][pl.ds(v * L, L)] = i3

    def fire(slot):
        pltpu.async_copy(pt_hbm.at[gidxs[slot]], rows[slot], gsems[slot])

    def drain(slot):
        pltpu.make_async_copy(pt_hbm.at[gidxs[slot]], rows[slot],
                              gsems[slot]).wait()

    def owait(slot):
        pltpu.make_async_copy(outs[slot],
                              out_hbm.at[pl.ds(base * EMB, BLK * EMB)],
                              osems[slot]).wait()

    def compute(slot):
        def gbody(g, carry):
            tvec = lax.iota(jnp.int32, L) + g * L
            tvec64 = tvec * EMB
            i3g = i3s[slot][pl.ds(g * L, L)]
            cbase = i3g * 32
            cv = [plsc.load_gather(ct_v, [cbase + _splat(c)])
                  for c in range(32)]
            for jj in range(16):
                j1, j2 = jj // 4, jj % 4
                pv = [plsc.load_gather(rows[slot],
                                       [tvec, _splat(j1 * 32 + j2 * 8 + r2)])
                      for r2 in range(8)]
                for j3 in range(4):
                    acc = pv[0] * cv[j3]
                    for r2 in range(1, 8):
                        acc = acc + pv[r2] * cv[r2 * 4 + j3]
                    plsc.store_scatter(outs[slot],
                                       [tvec64 + _splat(jj * 4 + j3)], acc)
            return carry
        lax.fori_loop(0, BLK // L, gbody, 0)

    make_idx(0, 0)
    fire(0)

    def outer(i, carry):
        for par in range(2):
            bb = 2 * i + par

            @pl.when(bb + 1 < NB)
            def _prefetch():
                make_idx(bb + 1, 1 - par)
                fire(1 - par)

            drain(par)

            @pl.when(bb >= 2)
            def _wait_out():
                owait(par)

            compute(par)
            pltpu.async_copy(
                outs[par],
                out_hbm.at[pl.ds((base + bb * BLK) * EMB, BLK * EMB)],
                osems[par])
        return carry

    lax.fori_loop(0, NB // 2, outer, 0)
    owait(0)
    owait(1)


def kernel(indices, core0, core1, core2):
    lhs = core0[0].reshape(100, 32)
    eye4 = jnp.eye(4, dtype=core1.dtype)
    rhs = (eye4[:, None, None, :, None, None]
           * core1[None, :, :, None, :, :]).reshape(32, 12800)
    pt = _pt_matmul(lhs, rhs).reshape(10000, 128)
    ct = jnp.transpose(core2[:, :, :, 0], (1, 0, 2)).reshape(3200)
    idx = indices.reshape(-1)
    out = _sc_lookup(idx, pt, ct)
    return out.reshape(BATCH, FIELDS, EMB)


# speedup vs baseline: 19.3020x; 2.0088x over previous
import functools

import jax
import jax.numpy as jnp
from jax import lax
from jax.experimental import pallas as pl
from jax.experimental.pallas import tpu as pltpu
from jax.experimental.pallas import tpu_sc as plsc

BATCH = 16384
FIELDS = 26
N = BATCH * FIELDS
EMB = 64
NC = 2
NS = 16
NW = NC * NS
TPW = N // NW
BLK = 128
NB = TPW // BLK
L = 16


def _pt_matmul(lhs, rhs):
    def body(l_ref, r_ref, o_ref):
        o_ref[...] = jnp.dot(l_ref[...], r_ref[...],
                             preferred_element_type=jnp.float32)
    return pl.pallas_call(
        body,
        out_shape=jax.ShapeDtypeStruct((100, 12800), jnp.float32),
    )(lhs, rhs)


def _splat(v):
    return jnp.full((L,), v, dtype=jnp.int32)


_mesh = plsc.VectorSubcoreMesh(core_axis_name="c", subcore_axis_name="s")


@functools.partial(
    pl.kernel,
    mesh=_mesh,
    compiler_params=pltpu.CompilerParams(needs_layout_passes=False),
    out_type=jax.ShapeDtypeStruct((N * EMB,), jnp.float32),
    scratch_types=[
        pltpu.VMEM((TPW,), jnp.int32),
        pltpu.VMEM((3304,), jnp.float32),
        pltpu.VMEM((BLK,), jnp.int32),
        pltpu.VMEM((BLK,), jnp.int32),
        pltpu.VMEM((BLK,), jnp.int32),
        pltpu.VMEM((BLK,), jnp.int32),
        pltpu.VMEM((BLK, 128), jnp.float32),
        pltpu.VMEM((BLK, 128), jnp.float32),
        pltpu.VMEM((BLK * EMB,), jnp.float32),
        pltpu.VMEM((BLK * EMB,), jnp.float32),
        pltpu.SemaphoreType.DMA,
        pltpu.SemaphoreType.DMA,
        pltpu.SemaphoreType.DMA,
        pltpu.SemaphoreType.DMA,
    ],
)
def _sc_lookup(idx_hbm, pt_hbm, ct_hbm, out_hbm,
               idx_v, ct_v, gidx0, gidx1, i30, i31, rows0, rows1,
               out0, out1, gsem0, gsem1, osem0, osem1):
    gidxs = (gidx0, gidx1)
    i3s = (i30, i31)
    rows = (rows0, rows1)
    outs = (out0, out1)
    gsems = (gsem0, gsem1)
    osems = (osem0, osem1)
    wid = lax.axis_index("s") * NC + lax.axis_index("c")
    base = wid * TPW

    pltpu.sync_copy(idx_hbm.at[pl.ds(base, TPW)], idx_v)
    pltpu.sync_copy(ct_hbm, ct_v)

    def make_idx(bb, slot):
        for v in range(BLK // L):
            iv = idx_v[pl.ds(bb * BLK + v * L, L)]
            f = iv.astype(jnp.float32) + 0.5
            i12 = (f * 0.01).astype(jnp.int32)
            i3 = iv - i12 * 100
            gidxs[slot][pl.ds(v * L, L)] = i12
            i3s[slot][pl.ds(v * L, L)] = i3

    def fire(slot):
        pltpu.async_copy(pt_hbm.at[gidxs[slot]], rows[slot], gsems[slot])

    def drain(slot):
        pltpu.make_async_copy(pt_hbm.at[gidxs[slot]], rows[slot],
                              gsems[slot]).wait()

    def owait(slot):
        pltpu.make_async_copy(outs[slot],
                              out_hbm.at[pl.ds(base * EMB, BLK * EMB)],
                              osems[slot]).wait()

    def compute(slot):
        iota = lax.iota(jnp.int32, L)
        jjadd = iota >> 3
        r2rot = [(iota + r) & 7 for r in range(8)]
        j3rot = [(iota + j) & 3 for j in range(4)]

        def gbody(g, carry):
            tvec = iota + g * L
            tvec64 = tvec * EMB
            i3g = i3s[slot][pl.ds(g * L, L)]
            cbase = i3g * 33
            crr = [cbase + (r2rot[r] << 2) for r in range(8)]
            cv = [[plsc.load_gather(ct_v, [crr[r] + j3rot[j]])
                   for j in range(4)] for r in range(8)]
            for jj in range(16):
                jjl = (jjadd + jj) & 15
                jj8 = jjl << 3
                jj4 = tvec64 + (jjl << 2)
                pv = [plsc.load_gather(rows[slot], [tvec, jj8 + r2rot[r]])
                      for r in range(8)]
                for j3 in range(4):
                    acc = pv[0] * cv[0][j3]
                    for r in range(1, 8):
                        acc = acc + pv[r] * cv[r][j3]
                    plsc.store_scatter(outs[slot], [jj4 + j3rot[j3]], acc)
            return carry
        lax.fori_loop(0, BLK // L, gbody, 0)

    make_idx(0, 0)
    fire(0)

    def outer(i, carry):
        for par in range(2):
            bb = 2 * i + par

            @pl.when(bb + 1 < NB)
            def _prefetch():
                make_idx(bb + 1, 1 - par)
                fire(1 - par)

            drain(par)

            @pl.when(bb >= 2)
            def _wait_out():
                owait(par)

            compute(par)
            pltpu.async_copy(
                outs[par],
                out_hbm.at[pl.ds((base + bb * BLK) * EMB, BLK * EMB)],
                osems[par])
        return carry

    lax.fori_loop(0, NB // 2, outer, 0)
    owait(0)
    owait(1)


def kernel(indices, core0, core1, core2):
    lhs = core0[0].reshape(100, 32)
    eye4 = jnp.eye(4, dtype=core1.dtype)
    rhs = (eye4[:, None, None, :, None, None]
           * core1[None, :, :, None, :, :]).reshape(32, 12800)
    pt = _pt_matmul(lhs, rhs).reshape(10000, 128)
    ct = jnp.transpose(core2[:, :, :, 0], (1, 0, 2)).reshape(100, 32)
    ct = jnp.pad(ct, ((0, 0), (0, 1))).reshape(3300)
    ct = jnp.pad(ct, (0, 4))
    idx = indices.reshape(-1)
    out = _sc_lookup(idx, pt, ct)
    return out.reshape(BATCH, FIELDS, EMB)
